# baseline (speedup 1.0000x reference)
import functools

import jax
import jax.numpy as jnp
from jax import lax
from jax.experimental import pallas as pl
from jax.experimental.pallas import tpu as pltpu
from jax.experimental.pallas import tpu_sc as plsc

BATCH = 4096
SEQ = 200
HIDDEN = 64
VOCAB = 1000000

_info = plsc.get_sparse_core_info()
NC, NS = _info.num_cores, _info.num_subcores
NW = NC * NS
B_PER_W = BATCH // NW
NBUF = 4
NOB = 2
GROUPS = SEQ // NBUF
L = 16


@functools.partial(
    pl.kernel,
    out_type=jax.ShapeDtypeStruct((SEQ, HIDDEN, BATCH), jnp.float32),
    mesh=plsc.VectorSubcoreMesh(core_axis_name="c", subcore_axis_name="s"),
    scratch_types=[
        pltpu.VMEM((SEQ, B_PER_W), jnp.int32),
        pltpu.VMEM((NBUF, B_PER_W), jnp.int32),
        pltpu.VMEM((NBUF, B_PER_W, 128), jnp.float32),
        pltpu.VMEM((NOB, HIDDEN, B_PER_W), jnp.float32),
        pltpu.SemaphoreType.DMA((NBUF,)),
        pltpu.SemaphoreType.DMA((NOB,)),
    ],
    compiler_params=pltpu.CompilerParams(needs_layout_passes=False),
)
def _gather_kernel(table_hbm, idx_hbm, out_hbm, idx_v, pv_v, pair_v, outb_v,
                   gsem, wsem):
    wid = lax.axis_index("s") * NC + lax.axis_index("c")
    pltpu.sync_copy(idx_hbm.at[wid], idx_v)

    def prep(s, b):
        for g in range(B_PER_W // L):
            v = idx_v[s, pl.ds(g * L, L)]
            pv_v[b, pl.ds(g * L, L)] = lax.shift_right_logical(v, 1)

    def gather(b):
        return pltpu.make_async_copy(
            table_hbm.at[pv_v.at[b]], pair_v.at[b], gsem.at[b])

    def writeback(s, ob):
        return pltpu.make_async_copy(
            outb_v.at[ob], out_hbm.at[s, :, pl.ds(wid * B_PER_W, B_PER_W)],
            wsem.at[ob])

    def transpose(s, b, ob):
        pair = pair_v.at[b]
        for g in range(B_PER_W // L):
            rows = lax.iota(jnp.int32, L) + (g * L)
            v = idx_v[s, pl.ds(g * L, L)]
            coloff = lax.shift_left(v & 1, 6)
            for h in range(HIDDEN):
                val = plsc.load_gather(pair, [rows, coloff + h])
                outb_v[ob, h, pl.ds(g * L, L)] = val

    for b in range(NBUF):
        prep(b, b)
        gather(b).start()

    def group(g, carry):
        for b in range(NBUF):
            s = g * NBUF + b
            ob = b & 1
            gather(b).wait()

            @pl.when(s >= NOB)
            def _():
                writeback(s - NOB, ob).wait()

            transpose(s, b, ob)
            writeback(s, ob).start()

            @pl.when(s + NBUF < SEQ)
            def _():
                prep(s + NBUF, b)
                gather(b).start()

        return carry

    lax.fori_loop(0, GROUPS, group, 0)
    writeback(SEQ - 2, 0).wait()
    writeback(SEQ - 1, 1).wait()


def kernel(inp_seq, inp_seq_len, embedding_table):
    del inp_seq_len
    table2 = embedding_table.reshape(VOCAB // 2, 128)
    idx_t = (
        inp_seq.astype(jnp.int32)
        .reshape(NW, B_PER_W, SEQ)
        .transpose(0, 2, 1)
    )
    out_t = _gather_kernel(table2, idx_t)
    return jnp.transpose(out_t, (2, 0, 1))
